# baseline (speedup 1.0000x reference)
import functools

import jax
import jax.numpy as jnp
from jax import lax
from jax.experimental import pallas as pl
from jax.experimental.pallas import tpu as pltpu
from jax.experimental.pallas import tpu_sc as plsc

NN = 100000
EE = 3200000
FF = 16
NC = 2
NS = 16
NW = NC * NS
NPAD = 100096
RPT = NPAD // NS
ZR = RPT // 34
CZ = RPT // 2
CH_A = 800
CH_C = 800
CH_E = 800

_f32 = jnp.float32
_i32 = jnp.int32


def _sc_mesh():
    return plsc.VectorSubcoreMesh(core_axis_name="c", subcore_axis_name="s")


def _zero_rows(buf, nrows):
    z16 = jnp.zeros((16,), _f32)

    def body(i, c):
        buf[i, :] = z16
        return c

    lax.fori_loop(0, nrows, body, 0)


def _fill_flat(buf, nvec, value):
    v16 = jnp.full((16,), value, _f32)

    def body(i, c):
        buf[pl.ds(i * 16, 16)] = v16
        return c

    lax.fori_loop(0, nvec, body, 0)


def _agg_body(srcs, dsts, eattr, sums, cnts,
              zrows, czero, idx_v, rows_v, ones_v, acc, csh):
    cid = lax.axis_index("c")
    sid = lax.axis_index("s")
    _zero_rows(zrows, ZR)
    _fill_flat(czero, CZ // 16, 0.0)
    _fill_flat(ones_v, CH_A // 16, 1.0)
    r0 = sid * RPT
    for j in range(34):
        pltpu.sync_copy(zrows, acc.at[pl.ds(r0 + j * ZR, ZR)])
    for j in range(2):
        pltpu.sync_copy(czero, csh.at[pl.ds(r0 + j * CZ, CZ)])
    plsc.subcore_barrier()

    e_base = sid * (EE // NS)

    def step(g, c):
        e0 = e_base + g * CH_A

        @pl.when(cid == 0)
        def _():
            pltpu.sync_copy(dsts.at[pl.ds(e0, CH_A)], idx_v)

        @pl.when(cid == 1)
        def _():
            pltpu.sync_copy(srcs.at[pl.ds(e0, CH_A)], idx_v)

        pltpu.sync_copy(eattr.at[pl.ds(e0, CH_A)], rows_v)
        pltpu.sync_copy(rows_v, acc.at[idx_v], add=True)
        pltpu.sync_copy(ones_v, csh.at[idx_v], add=True)
        return c

    lax.fori_loop(0, (EE // NS) // CH_A, step, 0)
    plsc.subcore_barrier()
    for j in range(34):
        pltpu.sync_copy(acc.at[pl.ds(r0 + j * ZR, ZR)], zrows)
        pltpu.sync_copy(zrows, sums.at[cid, pl.ds(r0 + j * ZR, ZR)])
    for j in range(2):
        pltpu.sync_copy(csh.at[pl.ds(r0 + j * CZ, CZ)], czero)
        pltpu.sync_copy(czero, cnts.at[pl.ds(cid * NPAD + r0 + j * CZ, CZ)])


_agg = pl.kernel(
    _agg_body,
    out_type=(jax.ShapeDtypeStruct((NC, NPAD, FF), _f32),
              jax.ShapeDtypeStruct((NC * NPAD,), _f32)),
    mesh=_sc_mesh(),
    compiler_params=pltpu.CompilerParams(use_tc_tiling_on_sc=False),
    scratch_types=[
        pltpu.VMEM((ZR, FF), _f32),
        pltpu.VMEM((CZ,), _f32),
        pltpu.VMEM((CH_A,), _i32),
        pltpu.VMEM((CH_A, FF), _f32),
        pltpu.VMEM((CH_A,), _f32),
        pltpu.VMEM_SHARED((NPAD, FF), _f32),
        pltpu.VMEM_SHARED((NPAD,), _f32),
    ],
)


def _spmm_body(srcs, dsts, h, parts,
               zrows, idx_s, idx_d, rows_v, acc, sem):
    cid = lax.axis_index("c")
    sid = lax.axis_index("s")
    _zero_rows(zrows, ZR)
    r0 = sid * RPT
    for j in range(34):
        pltpu.sync_copy(zrows, acc.at[pl.ds(r0 + j * ZR, ZR)])
    plsc.subcore_barrier()

    wid = cid * NS + sid
    e_base = wid * (EE // NW)

    def step(g, c):
        e0 = e_base + g * CH_C
        pltpu.sync_copy(srcs.at[pl.ds(e0, CH_C)], idx_s)
        pltpu.sync_copy(dsts.at[pl.ds(e0, CH_C)], idx_d)
        pltpu.async_copy(h.at[idx_s], rows_v, sem).wait()
        pltpu.sync_copy(rows_v, acc.at[idx_d], add=True)
        return c

    lax.fori_loop(0, (EE // NW) // CH_C, step, 0)
    plsc.subcore_barrier()
    for j in range(34):
        pltpu.sync_copy(acc.at[pl.ds(r0 + j * ZR, ZR)], zrows)
        pltpu.sync_copy(zrows, parts.at[cid, pl.ds(r0 + j * ZR, ZR)])


_spmm = pl.kernel(
    _spmm_body,
    out_type=jax.ShapeDtypeStruct((NC, NPAD, FF), _f32),
    mesh=_sc_mesh(),
    compiler_params=pltpu.CompilerParams(use_tc_tiling_on_sc=False),
    scratch_types=[
        pltpu.VMEM((ZR, FF), _f32),
        pltpu.VMEM((CH_C,), _i32),
        pltpu.VMEM((CH_C,), _i32),
        pltpu.VMEM((CH_C, FF), _f32),
        pltpu.VMEM_SHARED((NPAD, FF), _f32),
        pltpu.SemaphoreType.DMA,
    ],
)


def _edge_body(ii, srcs, dsts, na, u, v, emb, logits,
               ii_v, s_v, d_v, rows2, ug, vg, lg, ustage, ush, vsh, sem):
    cid = lax.axis_index("c")
    sid = lax.axis_index("s")
    r0 = sid * RPT
    pltpu.sync_copy(u.at[pl.ds(r0, RPT)], ustage)
    pltpu.sync_copy(ustage, ush.at[pl.ds(r0, RPT)])
    pltpu.sync_copy(v.at[pl.ds(r0, RPT)], ustage)
    pltpu.sync_copy(ustage, vsh.at[pl.ds(r0, RPT)])
    plsc.subcore_barrier()

    wid = cid * NS + sid
    e_base = wid * (EE // NW)

    def step(g, c):
        e0 = e_base + g * CH_E
        pltpu.sync_copy(ii.at[pl.ds(2 * e0, 2 * CH_E)], ii_v)
        pltpu.sync_copy(srcs.at[pl.ds(e0, CH_E)], s_v)
        pltpu.sync_copy(dsts.at[pl.ds(e0, CH_E)], d_v)
        pltpu.async_copy(na.at[ii_v], rows2, sem).wait()
        pltpu.async_copy(ush.at[s_v], ug, sem).wait()
        pltpu.async_copy(vsh.at[d_v], vg, sem).wait()
        pltpu.sync_copy(rows2, emb.at[pl.ds(2 * e0, 2 * CH_E)])

        def cstep(j, cc):
            lg[pl.ds(j * 16, 16)] = ug[pl.ds(j * 16, 16)] + vg[pl.ds(j * 16, 16)]
            return cc

        lax.fori_loop(0, CH_E // 16, cstep, 0)
        pltpu.sync_copy(lg, logits.at[pl.ds(e0, CH_E)])
        return c

    lax.fori_loop(0, (EE // NW) // CH_E, step, 0)


_edge = pl.kernel(
    _edge_body,
    out_type=(jax.ShapeDtypeStruct((2 * EE, FF), _f32),
              jax.ShapeDtypeStruct((EE,), _f32)),
    mesh=_sc_mesh(),
    compiler_params=pltpu.CompilerParams(use_tc_tiling_on_sc=False),
    scratch_types=[
        pltpu.VMEM((2 * CH_E,), _i32),
        pltpu.VMEM((CH_E,), _i32),
        pltpu.VMEM((CH_E,), _i32),
        pltpu.VMEM((2 * CH_E, FF), _f32),
        pltpu.VMEM((CH_E,), _f32),
        pltpu.VMEM((CH_E,), _f32),
        pltpu.VMEM((CH_E,), _f32),
        pltpu.VMEM((RPT,), _f32),
        pltpu.VMEM_SHARED((NPAD,), _f32),
        pltpu.VMEM_SHARED((NPAD,), _f32),
        pltpu.SemaphoreType.DMA,
    ],
)


BR = 2000
GRID = NN // BR


def _b0_kernel(sd, ss, cd, cs, wa, wb, b2, h0, ea_out):
    cd_ = jnp.maximum(cd[0], 1.0)
    cs_ = jnp.maximum(cs[0], 1.0)
    na = sd[0] / cd_
    ea = ss[0] / cs_
    h = jnp.dot(na, wa[...], preferred_element_type=_f32)
    h = h + jnp.dot(ea, wb[...], preferred_element_type=_f32)
    h0[...] = jnp.maximum(h + b2[0:1, :], 0.0)
    ea_out[...] = ea


def _db_kernel(p0, p1, cd, h0, wna, wnb, bn2, wea, web, be2, ea, h1_out):
    cd_ = jnp.maximum(cd[0], 1.0)
    nbr = (p0[0] + p1[0]) / cd_
    na1 = jnp.dot(h0[...], wna[...], preferred_element_type=_f32)
    na1 = na1 + jnp.dot(nbr, wnb[...], preferred_element_type=_f32)
    na1 = jnp.maximum(na1 + bn2[0:1, :], 0.0)
    h1 = jnp.dot(na1, wea[...], preferred_element_type=_f32)
    h1 = h1 + jnp.dot(ea[...], web[...], preferred_element_type=_f32)
    h1_out[...] = jnp.maximum(h1 + be2[0:1, :], 0.0)


def _d1_kernel(p0, p1, cd, h1, wna, wnb, bn2, wuv, buv, na2_out, u_out, v_out):
    cd_ = jnp.maximum(cd[0], 1.0)
    nbr = (p0[0] + p1[0]) / cd_
    na2 = jnp.dot(h1[...], wna[...], preferred_element_type=_f32)
    na2 = na2 + jnp.dot(nbr, wnb[...], preferred_element_type=_f32)
    na2 = jnp.maximum(na2 + bn2[0:1, :], 0.0)
    na2_out[...] = na2
    uv = jnp.dot(na2, wuv[...], preferred_element_type=_f32) + buv[0:1, :]
    u_out[...] = uv[:, 0:1]
    v_out[...] = uv[:, 1:2]


def _node_spec():
    return pl.BlockSpec((1, BR, FF), lambda i: (0, i, 0))


def _node_spec1():
    return pl.BlockSpec((1, BR, FF), lambda i: (1, i, 0))


def _cnt_spec(row):
    return pl.BlockSpec((1, BR, 1), lambda i, r=row: (r, i, 0))


def _w_spec(shape):
    return pl.BlockSpec(shape, lambda i: tuple(0 for _ in shape))


def _out_spec(width):
    return pl.BlockSpec((BR, width), lambda i: (i, 0))


_b0_call = pl.pallas_call(
    _b0_kernel,
    grid=(GRID,),
    in_specs=[_node_spec(), _node_spec1(), _cnt_spec(0), _cnt_spec(1),
              _w_spec((FF, FF)), _w_spec((FF, FF)), _w_spec((8, FF))],
    out_specs=[_out_spec(FF), _out_spec(FF)],
    out_shape=(jax.ShapeDtypeStruct((NN, FF), _f32),
               jax.ShapeDtypeStruct((NN, FF), _f32)),
)

_db_call = pl.pallas_call(
    _db_kernel,
    grid=(GRID,),
    in_specs=[_node_spec(), _node_spec1(), _cnt_spec(0), _out_spec(FF),
              _w_spec((FF, FF)), _w_spec((FF, FF)), _w_spec((8, FF)),
              _w_spec((FF, FF)), _w_spec((FF, FF)), _w_spec((8, FF)),
              _out_spec(FF)],
    out_specs=[_out_spec(FF)],
    out_shape=(jax.ShapeDtypeStruct((NN, FF), _f32),),
)

_d1_call = pl.pallas_call(
    _d1_kernel,
    grid=(GRID,),
    in_specs=[_node_spec(), _node_spec1(), _cnt_spec(0), _out_spec(FF),
              _w_spec((FF, FF)), _w_spec((FF, FF)), _w_spec((8, FF)),
              _w_spec((FF, 2)), _w_spec((8, 2))],
    out_specs=[_out_spec(FF), _out_spec(1), _out_spec(1)],
    out_shape=(jax.ShapeDtypeStruct((NN, FF), _f32),
               jax.ShapeDtypeStruct((NN, 1), _f32),
               jax.ShapeDtypeStruct((NN, 1), _f32)),
)


def kernel(edge_attr, edge_index, We0, be0, Wn0, bn0, We1, be1, Wn1, bn1, Wf, bf):
    srcs = edge_index[0]
    dsts = edge_index[1]
    sums, cnts = _agg(srcs, dsts, edge_attr)
    cnts3 = cnts.reshape(NC, NPAD, 1)

    def split_t(W):
        return W[:, :FF].T, W[:, FF:].T

    def b2(b):
        return jnp.broadcast_to(b.reshape(1, -1), (8, b.shape[0]))

    WeA0, WeB0 = split_t(We0)
    WnA0, WnB0 = split_t(Wn0)
    WeA1, WeB1 = split_t(We1)
    WnA1, WnB1 = split_t(Wn1)
    Wuv = jnp.stack([Wf[0, :FF], Wf[0, FF:]], axis=1)
    buv = jnp.broadcast_to(jnp.stack([bf[0], jnp.zeros((), _f32)]).reshape(1, 2), (8, 2))

    h0, eagg = _b0_call(sums, sums, cnts3, cnts3, WeA0, WeB0, b2(be0))
    parts0 = _spmm(srcs, dsts, h0)
    (h1,) = _db_call(parts0, parts0, cnts3, h0, WnA0, WnB0, b2(bn0),
                     WeA1, WeB1, b2(be1), eagg)
    parts1 = _spmm(srcs, dsts, h1)
    na2, u, v = _d1_call(parts1, parts1, cnts3, h1, WnA1, WnB1, b2(bn1),
                         Wuv, buv)

    pad = jnp.zeros((NPAD - NN,), _f32)
    u_p = jnp.concatenate([u.reshape(-1), pad])
    v_p = jnp.concatenate([v.reshape(-1), pad])

    ii = jnp.transpose(edge_index).reshape(-1)
    emb_flat, logits = _edge(ii, srcs, dsts, na2, u_p, v_p)
    edge_embeddings = emb_flat.reshape(EE, 2 * FF)
    return logits, edge_embeddings, na2


# speedup vs baseline: 1.2156x; 1.2156x over previous
import functools

import jax
import jax.numpy as jnp
from jax import lax
from jax.experimental import pallas as pl
from jax.experimental.pallas import tpu as pltpu
from jax.experimental.pallas import tpu_sc as plsc

NN = 100000
EE = 3200000
FF = 16
NC = 2
NS = 16
NW = NC * NS
NPAD = 100096
RPT = NPAD // NS
ZR = RPT // 34
CZ = RPT // 2
CH_A = 800
CH_C = 800
CH_E = 800

_f32 = jnp.float32
_i32 = jnp.int32


def _sc_mesh():
    return plsc.VectorSubcoreMesh(core_axis_name="c", subcore_axis_name="s")


def _zero_rows(buf, nrows):
    z16 = jnp.zeros((16,), _f32)

    def body(i, c):
        buf[i, :] = z16
        return c

    lax.fori_loop(0, nrows, body, 0)


def _fill_flat(buf, nvec, value):
    v16 = jnp.full((16,), value, _f32)

    def body(i, c):
        buf[pl.ds(i * 16, 16)] = v16
        return c

    lax.fori_loop(0, nvec, body, 0)


def _agg_body(srcs, dsts, eattr, sums, cnts,
              zrows, czero, idx_v, rows_v, ones_v, acc, csh):
    cid = lax.axis_index("c")
    sid = lax.axis_index("s")
    _zero_rows(zrows, ZR)
    _fill_flat(czero, CZ // 16, 0.0)
    _fill_flat(ones_v, CH_A // 16, 1.0)
    r0 = sid * RPT
    for j in range(34):
        pltpu.sync_copy(zrows, acc.at[pl.ds(r0 + j * ZR, ZR)])
    for j in range(2):
        pltpu.sync_copy(czero, csh.at[pl.ds(r0 + j * CZ, CZ)])
    plsc.subcore_barrier()

    e_base = sid * (EE // NS)

    def step(g, c):
        e0 = e_base + g * CH_A

        @pl.when(cid == 0)
        def _():
            pltpu.sync_copy(dsts.at[pl.ds(e0, CH_A)], idx_v)

        @pl.when(cid == 1)
        def _():
            pltpu.sync_copy(srcs.at[pl.ds(e0, CH_A)], idx_v)

        pltpu.sync_copy(eattr.at[pl.ds(e0, CH_A)], rows_v)
        pltpu.sync_copy(rows_v, acc.at[idx_v], add=True)
        pltpu.sync_copy(ones_v, csh.at[idx_v], add=True)
        return c

    lax.fori_loop(0, (EE // NS) // CH_A, step, 0)
    plsc.subcore_barrier()
    for j in range(34):
        pltpu.sync_copy(acc.at[pl.ds(r0 + j * ZR, ZR)], zrows)
        pltpu.sync_copy(zrows, sums.at[cid, pl.ds(r0 + j * ZR, ZR)])
    for j in range(2):
        pltpu.sync_copy(csh.at[pl.ds(r0 + j * CZ, CZ)], czero)
        pltpu.sync_copy(czero, cnts.at[pl.ds(cid * NPAD + r0 + j * CZ, CZ)])


_agg = pl.kernel(
    _agg_body,
    out_type=(jax.ShapeDtypeStruct((NC, NPAD, FF), _f32),
              jax.ShapeDtypeStruct((NC * NPAD,), _f32)),
    mesh=_sc_mesh(),
    compiler_params=pltpu.CompilerParams(use_tc_tiling_on_sc=False),
    scratch_types=[
        pltpu.VMEM((ZR, FF), _f32),
        pltpu.VMEM((CZ,), _f32),
        pltpu.VMEM((CH_A,), _i32),
        pltpu.VMEM((CH_A, FF), _f32),
        pltpu.VMEM((CH_A,), _f32),
        pltpu.VMEM_SHARED((NPAD, FF), _f32),
        pltpu.VMEM_SHARED((NPAD,), _f32),
    ],
)


def _spmm_body(srcs, dsts, h, parts,
               zrows, idx_s, idx_d, rows_v, acc, sem):
    cid = lax.axis_index("c")
    sid = lax.axis_index("s")
    _zero_rows(zrows, ZR)
    r0 = sid * RPT
    for j in range(34):
        pltpu.sync_copy(zrows, acc.at[pl.ds(r0 + j * ZR, ZR)])
    plsc.subcore_barrier()

    wid = cid * NS + sid
    e_base = wid * (EE // NW)

    def step(g, c):
        e0 = e_base + g * CH_C
        pltpu.sync_copy(srcs.at[pl.ds(e0, CH_C)], idx_s)
        pltpu.sync_copy(dsts.at[pl.ds(e0, CH_C)], idx_d)
        pltpu.async_copy(h.at[idx_s], rows_v, sem).wait()
        pltpu.sync_copy(rows_v, acc.at[idx_d], add=True)
        return c

    lax.fori_loop(0, (EE // NW) // CH_C, step, 0)
    plsc.subcore_barrier()
    for j in range(34):
        pltpu.sync_copy(acc.at[pl.ds(r0 + j * ZR, ZR)], zrows)
        pltpu.sync_copy(zrows, parts.at[cid, pl.ds(r0 + j * ZR, ZR)])


_spmm = pl.kernel(
    _spmm_body,
    out_type=jax.ShapeDtypeStruct((NC, NPAD, FF), _f32),
    mesh=_sc_mesh(),
    compiler_params=pltpu.CompilerParams(use_tc_tiling_on_sc=False),
    scratch_types=[
        pltpu.VMEM((ZR, FF), _f32),
        pltpu.VMEM((CH_C,), _i32),
        pltpu.VMEM((CH_C,), _i32),
        pltpu.VMEM((CH_C, FF), _f32),
        pltpu.VMEM_SHARED((NPAD, FF), _f32),
        pltpu.SemaphoreType.DMA,
    ],
)


def _edge_body(srcs, dsts, na, u, v, embA, embB, logits,
               s_v, d_v, rowsA, rowsB, ug, vg, lg, ustage, ush, vsh, sem):
    cid = lax.axis_index("c")
    sid = lax.axis_index("s")
    r0 = sid * RPT
    pltpu.sync_copy(u.at[pl.ds(r0, RPT)], ustage)
    pltpu.sync_copy(ustage, ush.at[pl.ds(r0, RPT)])
    pltpu.sync_copy(v.at[pl.ds(r0, RPT)], ustage)
    pltpu.sync_copy(ustage, vsh.at[pl.ds(r0, RPT)])
    plsc.subcore_barrier()

    wid = cid * NS + sid
    e_base = wid * (EE // NW)

    def step(g, c):
        e0 = e_base + g * CH_E
        pltpu.sync_copy(srcs.at[pl.ds(e0, CH_E)], s_v)
        pltpu.sync_copy(dsts.at[pl.ds(e0, CH_E)], d_v)
        pltpu.async_copy(na.at[s_v], rowsA, sem).wait()
        pltpu.async_copy(na.at[d_v], rowsB, sem).wait()
        pltpu.sync_copy(rowsA, embA.at[pl.ds(e0, CH_E)])
        pltpu.sync_copy(rowsB, embB.at[pl.ds(e0, CH_E)])
        pltpu.async_copy(ush.at[s_v], ug, sem).wait()
        pltpu.async_copy(vsh.at[d_v], vg, sem).wait()

        def cstep(j, cc):
            lg[pl.ds(j * 16, 16)] = ug[pl.ds(j * 16, 16)] + vg[pl.ds(j * 16, 16)]
            return cc

        lax.fori_loop(0, CH_E // 16, cstep, 0)
        pltpu.sync_copy(lg, logits.at[pl.ds(e0, CH_E)])
        return c

    lax.fori_loop(0, (EE // NW) // CH_E, step, 0)


_edge = pl.kernel(
    _edge_body,
    out_type=(jax.ShapeDtypeStruct((EE, FF), _f32),
              jax.ShapeDtypeStruct((EE, FF), _f32),
              jax.ShapeDtypeStruct((EE,), _f32)),
    mesh=_sc_mesh(),
    compiler_params=pltpu.CompilerParams(use_tc_tiling_on_sc=False),
    scratch_types=[
        pltpu.VMEM((CH_E,), _i32),
        pltpu.VMEM((CH_E,), _i32),
        pltpu.VMEM((CH_E, FF), _f32),
        pltpu.VMEM((CH_E, FF), _f32),
        pltpu.VMEM((CH_E,), _f32),
        pltpu.VMEM((CH_E,), _f32),
        pltpu.VMEM((CH_E,), _f32),
        pltpu.VMEM((RPT,), _f32),
        pltpu.VMEM_SHARED((NPAD,), _f32),
        pltpu.VMEM_SHARED((NPAD,), _f32),
        pltpu.SemaphoreType.DMA,
    ],
)


BR = 2000
GRID = NN // BR


def _b0_kernel(sd, ss, cd, cs, wa, wb, b2, h0, ea_out):
    cd_ = jnp.maximum(cd[0], 1.0)
    cs_ = jnp.maximum(cs[0], 1.0)
    na = sd[0] / cd_
    ea = ss[0] / cs_
    h = jnp.dot(na, wa[...], preferred_element_type=_f32)
    h = h + jnp.dot(ea, wb[...], preferred_element_type=_f32)
    h0[...] = jnp.maximum(h + b2[0:1, :], 0.0)
    ea_out[...] = ea


def _db_kernel(p0, p1, cd, h0, wna, wnb, bn2, wea, web, be2, ea, h1_out):
    cd_ = jnp.maximum(cd[0], 1.0)
    nbr = (p0[0] + p1[0]) / cd_
    na1 = jnp.dot(h0[...], wna[...], preferred_element_type=_f32)
    na1 = na1 + jnp.dot(nbr, wnb[...], preferred_element_type=_f32)
    na1 = jnp.maximum(na1 + bn2[0:1, :], 0.0)
    h1 = jnp.dot(na1, wea[...], preferred_element_type=_f32)
    h1 = h1 + jnp.dot(ea[...], web[...], preferred_element_type=_f32)
    h1_out[...] = jnp.maximum(h1 + be2[0:1, :], 0.0)


def _d1_kernel(p0, p1, cd, h1, wna, wnb, bn2, wuv, buv, na2_out, u_out, v_out):
    cd_ = jnp.maximum(cd[0], 1.0)
    nbr = (p0[0] + p1[0]) / cd_
    na2 = jnp.dot(h1[...], wna[...], preferred_element_type=_f32)
    na2 = na2 + jnp.dot(nbr, wnb[...], preferred_element_type=_f32)
    na2 = jnp.maximum(na2 + bn2[0:1, :], 0.0)
    na2_out[...] = na2
    uv = jnp.dot(na2, wuv[...], preferred_element_type=_f32) + buv[0:1, :]
    u_out[...] = uv[:, 0:1]
    v_out[...] = uv[:, 1:2]


def _node_spec():
    return pl.BlockSpec((1, BR, FF), lambda i: (0, i, 0))


def _node_spec1():
    return pl.BlockSpec((1, BR, FF), lambda i: (1, i, 0))


def _cnt_spec(row):
    return pl.BlockSpec((1, BR, 1), lambda i, r=row: (r, i, 0))


def _w_spec(shape):
    return pl.BlockSpec(shape, lambda i: tuple(0 for _ in shape))


def _out_spec(width):
    return pl.BlockSpec((BR, width), lambda i: (i, 0))


_b0_call = pl.pallas_call(
    _b0_kernel,
    grid=(GRID,),
    in_specs=[_node_spec(), _node_spec1(), _cnt_spec(0), _cnt_spec(1),
              _w_spec((FF, FF)), _w_spec((FF, FF)), _w_spec((8, FF))],
    out_specs=[_out_spec(FF), _out_spec(FF)],
    out_shape=(jax.ShapeDtypeStruct((NN, FF), _f32),
               jax.ShapeDtypeStruct((NN, FF), _f32)),
)

_db_call = pl.pallas_call(
    _db_kernel,
    grid=(GRID,),
    in_specs=[_node_spec(), _node_spec1(), _cnt_spec(0), _out_spec(FF),
              _w_spec((FF, FF)), _w_spec((FF, FF)), _w_spec((8, FF)),
              _w_spec((FF, FF)), _w_spec((FF, FF)), _w_spec((8, FF)),
              _out_spec(FF)],
    out_specs=[_out_spec(FF)],
    out_shape=(jax.ShapeDtypeStruct((NN, FF), _f32),),
)

_d1_call = pl.pallas_call(
    _d1_kernel,
    grid=(GRID,),
    in_specs=[_node_spec(), _node_spec1(), _cnt_spec(0), _out_spec(FF),
              _w_spec((FF, FF)), _w_spec((FF, FF)), _w_spec((8, FF)),
              _w_spec((FF, 2)), _w_spec((8, 2))],
    out_specs=[_out_spec(FF), _out_spec(1), _out_spec(1)],
    out_shape=(jax.ShapeDtypeStruct((NN, FF), _f32),
               jax.ShapeDtypeStruct((NN, 1), _f32),
               jax.ShapeDtypeStruct((NN, 1), _f32)),
)


BC = 3200
TGRID = EE // BC


def _emb_tpose_kernel(ea_, eb_, et_out):
    at = jnp.transpose(ea_[...], (1, 0))
    bt = jnp.transpose(eb_[...], (1, 0))
    et_out[...] = jnp.concatenate([at, bt], axis=0)


_emb_tpose = pl.pallas_call(
    _emb_tpose_kernel,
    grid=(TGRID,),
    in_specs=[pl.BlockSpec((BC, FF), lambda i: (i, 0)),
              pl.BlockSpec((BC, FF), lambda i: (i, 0))],
    out_specs=[pl.BlockSpec((2 * FF, BC), lambda i: (0, i))],
    out_shape=(jax.ShapeDtypeStruct((2 * FF, EE), _f32),),
)


def kernel(edge_attr, edge_index, We0, be0, Wn0, bn0, We1, be1, Wn1, bn1, Wf, bf):
    srcs = edge_index[0]
    dsts = edge_index[1]
    sums, cnts = _agg(srcs, dsts, edge_attr)
    cnts3 = cnts.reshape(NC, NPAD, 1)

    def split_t(W):
        return W[:, :FF].T, W[:, FF:].T

    def b2(b):
        return jnp.broadcast_to(b.reshape(1, -1), (8, b.shape[0]))

    WeA0, WeB0 = split_t(We0)
    WnA0, WnB0 = split_t(Wn0)
    WeA1, WeB1 = split_t(We1)
    WnA1, WnB1 = split_t(Wn1)
    Wuv = jnp.stack([Wf[0, :FF], Wf[0, FF:]], axis=1)
    buv = jnp.broadcast_to(jnp.stack([bf[0], jnp.zeros((), _f32)]).reshape(1, 2), (8, 2))

    h0, eagg = _b0_call(sums, sums, cnts3, cnts3, WeA0, WeB0, b2(be0))
    parts0 = _spmm(srcs, dsts, h0)
    (h1,) = _db_call(parts0, parts0, cnts3, h0, WnA0, WnB0, b2(bn0),
                     WeA1, WeB1, b2(be1), eagg)
    parts1 = _spmm(srcs, dsts, h1)
    na2, u, v = _d1_call(parts1, parts1, cnts3, h1, WnA1, WnB1, b2(bn1),
                         Wuv, buv)

    pad = jnp.zeros((NPAD - NN,), _f32)
    u_p = jnp.concatenate([u.reshape(-1), pad])
    v_p = jnp.concatenate([v.reshape(-1), pad])

    embA, embB, logits = _edge(srcs, dsts, na2, u_p, v_p)
    (emb_t,) = _emb_tpose(embA, embB)
    edge_embeddings = jnp.transpose(emb_t, (1, 0))
    return logits, edge_embeddings, na2


# speedup vs baseline: 1.2806x; 1.0535x over previous
import functools

import jax
import jax.numpy as jnp
from jax import lax
from jax.experimental import pallas as pl
from jax.experimental.pallas import tpu as pltpu
from jax.experimental.pallas import tpu_sc as plsc

NN = 100000
EE = 3200000
FF = 16
NC = 2
NS = 16
NW = NC * NS
NPAD = 100096
RPT = NPAD // NS
ZR = RPT // 34
CZ = RPT // 2
CH_A = 400
CH_C = 400
CH_E = 800

_f32 = jnp.float32
_i32 = jnp.int32


def _sc_mesh():
    return plsc.VectorSubcoreMesh(core_axis_name="c", subcore_axis_name="s")


def _zero_rows(buf, nrows):
    z16 = jnp.zeros((16,), _f32)

    def body(i, c):
        buf[i, :] = z16
        return c

    lax.fori_loop(0, nrows, body, 0)


def _fill_flat(buf, nvec, value):
    v16 = jnp.full((16,), value, _f32)

    def body(i, c):
        buf[pl.ds(i * 16, 16)] = v16
        return c

    lax.fori_loop(0, nvec, body, 0)


def _agg_body(srcs, dsts, eattr, sums, cnts,
              zrows, czero, idx_v, rows_v, ones_v, acc, csh, lsem):
    cid = lax.axis_index("c")
    sid = lax.axis_index("s")
    _zero_rows(zrows, ZR)
    _fill_flat(czero, CZ // 16, 0.0)
    _fill_flat(ones_v, CH_A // 16, 1.0)
    r0 = sid * RPT
    for j in range(34):
        pltpu.sync_copy(zrows, acc.at[pl.ds(r0 + j * ZR, ZR)])
    for j in range(2):
        pltpu.sync_copy(czero, csh.at[pl.ds(r0 + j * CZ, CZ)])
    plsc.subcore_barrier()

    e_base = sid * (EE // NS)
    NG_A = (EE // NS) // CH_A

    def issue_loads(g, b):
        e0 = e_base + g * CH_A

        @pl.when(cid == 0)
        def _():
            pltpu.async_copy(dsts.at[pl.ds(e0, CH_A)], idx_v.at[b], lsem.at[b])

        @pl.when(cid == 1)
        def _():
            pltpu.async_copy(srcs.at[pl.ds(e0, CH_A)], idx_v.at[b], lsem.at[b])

        pltpu.async_copy(eattr.at[pl.ds(e0, CH_A)], rows_v.at[b], lsem.at[b])

    issue_loads(0, 0)
    issue_loads(1, 1)

    def step(g, c):
        b = lax.rem(g, 2)
        pltpu.make_async_copy(srcs.at[pl.ds(0, CH_A)], idx_v.at[b],
                              lsem.at[b]).wait()
        pltpu.make_async_copy(eattr.at[pl.ds(0, CH_A)], rows_v.at[b],
                              lsem.at[b]).wait()
        pltpu.sync_copy(rows_v.at[b], acc.at[idx_v.at[b]], add=True)
        pltpu.sync_copy(ones_v, csh.at[idx_v.at[b]], add=True)

        @pl.when(g + 2 < NG_A)
        def _():
            issue_loads(g + 2, b)

        return c

    lax.fori_loop(0, NG_A, step, 0)
    plsc.subcore_barrier()
    for j in range(34):
        pltpu.sync_copy(acc.at[pl.ds(r0 + j * ZR, ZR)], zrows)
        pltpu.sync_copy(zrows, sums.at[cid, pl.ds(r0 + j * ZR, ZR)])
    for j in range(2):
        pltpu.sync_copy(csh.at[pl.ds(r0 + j * CZ, CZ)], czero)
        pltpu.sync_copy(czero, cnts.at[pl.ds(cid * NPAD + r0 + j * CZ, CZ)])


_agg = pl.kernel(
    _agg_body,
    out_type=(jax.ShapeDtypeStruct((NC, NPAD, FF), _f32),
              jax.ShapeDtypeStruct((NC * NPAD,), _f32)),
    mesh=_sc_mesh(),
    compiler_params=pltpu.CompilerParams(use_tc_tiling_on_sc=False),
    scratch_types=[
        pltpu.VMEM((ZR, FF), _f32),
        pltpu.VMEM((CZ,), _f32),
        pltpu.VMEM((2, CH_A), _i32),
        pltpu.VMEM((2, CH_A, FF), _f32),
        pltpu.VMEM((CH_A,), _f32),
        pltpu.VMEM_SHARED((NPAD, FF), _f32),
        pltpu.VMEM_SHARED((NPAD,), _f32),
        pltpu.SemaphoreType.DMA((2,)),
    ],
)


def _spmm_body(srcs, dsts, h, parts,
               zrows, idx_s, idx_d, rows_v, acc, sem):
    cid = lax.axis_index("c")
    sid = lax.axis_index("s")
    _zero_rows(zrows, ZR)
    r0 = sid * RPT
    for j in range(34):
        pltpu.sync_copy(zrows, acc.at[pl.ds(r0 + j * ZR, ZR)])
    plsc.subcore_barrier()

    wid = cid * NS + sid
    e_base = wid * (EE // NW)
    NG_C = (EE // NW) // CH_C

    def load_idx(g, b):
        e0 = e_base + g * CH_C
        pltpu.sync_copy(srcs.at[pl.ds(e0, CH_C)], idx_s.at[b])
        pltpu.sync_copy(dsts.at[pl.ds(e0, CH_C)], idx_d.at[b])

    load_idx(0, 0)
    pltpu.async_copy(h.at[idx_s.at[0]], rows_v.at[0], sem.at[0])
    load_idx(1, 1)

    def step(g, c):
        b = lax.rem(g, 2)
        ob = 1 - b

        @pl.when(g + 1 < NG_C)
        def _():
            pltpu.async_copy(h.at[idx_s.at[ob]], rows_v.at[ob], sem.at[ob])

        pltpu.make_async_copy(h.at[idx_s.at[b]], rows_v.at[b],
                              sem.at[b]).wait()
        pltpu.sync_copy(rows_v.at[b], acc.at[idx_d.at[b]], add=True)

        @pl.when(g + 2 < NG_C)
        def _():
            load_idx(g + 2, b)

        return c

    lax.fori_loop(0, NG_C, step, 0)
    plsc.subcore_barrier()
    for j in range(34):
        pltpu.sync_copy(acc.at[pl.ds(r0 + j * ZR, ZR)], zrows)
        pltpu.sync_copy(zrows, parts.at[cid, pl.ds(r0 + j * ZR, ZR)])


_spmm = pl.kernel(
    _spmm_body,
    out_type=jax.ShapeDtypeStruct((NC, NPAD, FF), _f32),
    mesh=_sc_mesh(),
    compiler_params=pltpu.CompilerParams(use_tc_tiling_on_sc=False),
    scratch_types=[
        pltpu.VMEM((ZR, FF), _f32),
        pltpu.VMEM((2, CH_C), _i32),
        pltpu.VMEM((2, CH_C), _i32),
        pltpu.VMEM((2, CH_C, FF), _f32),
        pltpu.VMEM_SHARED((NPAD, FF), _f32),
        pltpu.SemaphoreType.DMA((2,)),
    ],
)


def _edge_body(srcs, dsts, na, u, v, embA, embB, logits,
               s_v, d_v, rowsA, rowsB, ug, vg, lg, ustage, ush, vsh, sem):
    cid = lax.axis_index("c")
    sid = lax.axis_index("s")
    r0 = sid * RPT
    pltpu.sync_copy(u.at[pl.ds(r0, RPT)], ustage)
    pltpu.sync_copy(ustage, ush.at[pl.ds(r0, RPT)])
    pltpu.sync_copy(v.at[pl.ds(r0, RPT)], ustage)
    pltpu.sync_copy(ustage, vsh.at[pl.ds(r0, RPT)])
    plsc.subcore_barrier()

    wid = cid * NS + sid
    e_base = wid * (EE // NW)

    def step(g, c):
        e0 = e_base + g * CH_E
        pltpu.sync_copy(srcs.at[pl.ds(e0, CH_E)], s_v)
        pltpu.sync_copy(dsts.at[pl.ds(e0, CH_E)], d_v)
        pltpu.async_copy(na.at[s_v], rowsA, sem).wait()
        pltpu.async_copy(na.at[d_v], rowsB, sem).wait()
        pltpu.sync_copy(rowsA, embA.at[pl.ds(e0, CH_E)])
        pltpu.sync_copy(rowsB, embB.at[pl.ds(e0, CH_E)])
        pltpu.async_copy(ush.at[s_v], ug, sem).wait()
        pltpu.async_copy(vsh.at[d_v], vg, sem).wait()

        def cstep(j, cc):
            lg[pl.ds(j * 16, 16)] = ug[pl.ds(j * 16, 16)] + vg[pl.ds(j * 16, 16)]
            return cc

        lax.fori_loop(0, CH_E // 16, cstep, 0)
        pltpu.sync_copy(lg, logits.at[pl.ds(e0, CH_E)])
        return c

    lax.fori_loop(0, (EE // NW) // CH_E, step, 0)


_edge = pl.kernel(
    _edge_body,
    out_type=(jax.ShapeDtypeStruct((EE, FF), _f32),
              jax.ShapeDtypeStruct((EE, FF), _f32),
              jax.ShapeDtypeStruct((EE,), _f32)),
    mesh=_sc_mesh(),
    compiler_params=pltpu.CompilerParams(use_tc_tiling_on_sc=False),
    scratch_types=[
        pltpu.VMEM((CH_E,), _i32),
        pltpu.VMEM((CH_E,), _i32),
        pltpu.VMEM((CH_E, FF), _f32),
        pltpu.VMEM((CH_E, FF), _f32),
        pltpu.VMEM((CH_E,), _f32),
        pltpu.VMEM((CH_E,), _f32),
        pltpu.VMEM((CH_E,), _f32),
        pltpu.VMEM((RPT,), _f32),
        pltpu.VMEM_SHARED((NPAD,), _f32),
        pltpu.VMEM_SHARED((NPAD,), _f32),
        pltpu.SemaphoreType.DMA,
    ],
)


BR = 2000
GRID = NN // BR


def _b0_kernel(sd, ss, cd, cs, wa, wb, b2, h0, ea_out):
    cd_ = jnp.maximum(cd[0], 1.0)
    cs_ = jnp.maximum(cs[0], 1.0)
    na = sd[0] / cd_
    ea = ss[0] / cs_
    h = jnp.dot(na, wa[...], preferred_element_type=_f32)
    h = h + jnp.dot(ea, wb[...], preferred_element_type=_f32)
    h0[...] = jnp.maximum(h + b2[0:1, :], 0.0)
    ea_out[...] = ea


def _db_kernel(p0, p1, cd, h0, wna, wnb, bn2, wea, web, be2, ea, h1_out):
    cd_ = jnp.maximum(cd[0], 1.0)
    nbr = (p0[0] + p1[0]) / cd_
    na1 = jnp.dot(h0[...], wna[...], preferred_element_type=_f32)
    na1 = na1 + jnp.dot(nbr, wnb[...], preferred_element_type=_f32)
    na1 = jnp.maximum(na1 + bn2[0:1, :], 0.0)
    h1 = jnp.dot(na1, wea[...], preferred_element_type=_f32)
    h1 = h1 + jnp.dot(ea[...], web[...], preferred_element_type=_f32)
    h1_out[...] = jnp.maximum(h1 + be2[0:1, :], 0.0)


def _d1_kernel(p0, p1, cd, h1, wna, wnb, bn2, wuv, buv, na2_out, u_out, v_out):
    cd_ = jnp.maximum(cd[0], 1.0)
    nbr = (p0[0] + p1[0]) / cd_
    na2 = jnp.dot(h1[...], wna[...], preferred_element_type=_f32)
    na2 = na2 + jnp.dot(nbr, wnb[...], preferred_element_type=_f32)
    na2 = jnp.maximum(na2 + bn2[0:1, :], 0.0)
    na2_out[...] = na2
    uv = jnp.dot(na2, wuv[...], preferred_element_type=_f32) + buv[0:1, :]
    u_out[...] = uv[:, 0:1]
    v_out[...] = uv[:, 1:2]


def _node_spec():
    return pl.BlockSpec((1, BR, FF), lambda i: (0, i, 0))


def _node_spec1():
    return pl.BlockSpec((1, BR, FF), lambda i: (1, i, 0))


def _cnt_spec(row):
    return pl.BlockSpec((1, BR, 1), lambda i, r=row: (r, i, 0))


def _w_spec(shape):
    return pl.BlockSpec(shape, lambda i: tuple(0 for _ in shape))


def _out_spec(width):
    return pl.BlockSpec((BR, width), lambda i: (i, 0))


_b0_call = pl.pallas_call(
    _b0_kernel,
    grid=(GRID,),
    in_specs=[_node_spec(), _node_spec1(), _cnt_spec(0), _cnt_spec(1),
              _w_spec((FF, FF)), _w_spec((FF, FF)), _w_spec((8, FF))],
    out_specs=[_out_spec(FF), _out_spec(FF)],
    out_shape=(jax.ShapeDtypeStruct((NN, FF), _f32),
               jax.ShapeDtypeStruct((NN, FF), _f32)),
)

_db_call = pl.pallas_call(
    _db_kernel,
    grid=(GRID,),
    in_specs=[_node_spec(), _node_spec1(), _cnt_spec(0), _out_spec(FF),
              _w_spec((FF, FF)), _w_spec((FF, FF)), _w_spec((8, FF)),
              _w_spec((FF, FF)), _w_spec((FF, FF)), _w_spec((8, FF)),
              _out_spec(FF)],
    out_specs=[_out_spec(FF)],
    out_shape=(jax.ShapeDtypeStruct((NN, FF), _f32),),
)

_d1_call = pl.pallas_call(
    _d1_kernel,
    grid=(GRID,),
    in_specs=[_node_spec(), _node_spec1(), _cnt_spec(0), _out_spec(FF),
              _w_spec((FF, FF)), _w_spec((FF, FF)), _w_spec((8, FF)),
              _w_spec((FF, 2)), _w_spec((8, 2))],
    out_specs=[_out_spec(FF), _out_spec(1), _out_spec(1)],
    out_shape=(jax.ShapeDtypeStruct((NN, FF), _f32),
               jax.ShapeDtypeStruct((NN, 1), _f32),
               jax.ShapeDtypeStruct((NN, 1), _f32)),
)


BC = 3200
TGRID = EE // BC


def _emb_tpose_kernel(ea_, eb_, et_out):
    at = jnp.transpose(ea_[...], (1, 0))
    bt = jnp.transpose(eb_[...], (1, 0))
    et_out[...] = jnp.concatenate([at, bt], axis=0)


_emb_tpose = pl.pallas_call(
    _emb_tpose_kernel,
    grid=(TGRID,),
    in_specs=[pl.BlockSpec((BC, FF), lambda i: (i, 0)),
              pl.BlockSpec((BC, FF), lambda i: (i, 0))],
    out_specs=[pl.BlockSpec((2 * FF, BC), lambda i: (0, i))],
    out_shape=(jax.ShapeDtypeStruct((2 * FF, EE), _f32),),
)


def kernel(edge_attr, edge_index, We0, be0, Wn0, bn0, We1, be1, Wn1, bn1, Wf, bf):
    srcs = edge_index[0]
    dsts = edge_index[1]
    sums, cnts = _agg(srcs, dsts, edge_attr)
    cnts3 = cnts.reshape(NC, NPAD, 1)

    def split_t(W):
        return W[:, :FF].T, W[:, FF:].T

    def b2(b):
        return jnp.broadcast_to(b.reshape(1, -1), (8, b.shape[0]))

    WeA0, WeB0 = split_t(We0)
    WnA0, WnB0 = split_t(Wn0)
    WeA1, WeB1 = split_t(We1)
    WnA1, WnB1 = split_t(Wn1)
    Wuv = jnp.stack([Wf[0, :FF], Wf[0, FF:]], axis=1)
    buv = jnp.broadcast_to(jnp.stack([bf[0], jnp.zeros((), _f32)]).reshape(1, 2), (8, 2))

    h0, eagg = _b0_call(sums, sums, cnts3, cnts3, WeA0, WeB0, b2(be0))
    parts0 = _spmm(srcs, dsts, h0)
    (h1,) = _db_call(parts0, parts0, cnts3, h0, WnA0, WnB0, b2(bn0),
                     WeA1, WeB1, b2(be1), eagg)
    parts1 = _spmm(srcs, dsts, h1)
    na2, u, v = _d1_call(parts1, parts1, cnts3, h1, WnA1, WnB1, b2(bn1),
                         Wuv, buv)

    pad = jnp.zeros((NPAD - NN,), _f32)
    u_p = jnp.concatenate([u.reshape(-1), pad])
    v_p = jnp.concatenate([v.reshape(-1), pad])

    embA, embB, logits = _edge(srcs, dsts, na2, u_p, v_p)
    (emb_t,) = _emb_tpose(embA, embB)
    edge_embeddings = jnp.transpose(emb_t, (1, 0))
    return logits, edge_embeddings, na2


# speedup vs baseline: 1.3546x; 1.0578x over previous
import functools

import jax
import jax.numpy as jnp
from jax import lax
from jax.experimental import pallas as pl
from jax.experimental.pallas import tpu as pltpu
from jax.experimental.pallas import tpu_sc as plsc

NN = 100000
EE = 3200000
FF = 16
NC = 2
NS = 16
NW = NC * NS
NPAD = 100096
RPT = NPAD // NS
ZR = RPT // 34
CZ = RPT // 2
CH_A = 400
CH_C = 400
CH_E = 800

_f32 = jnp.float32
_i32 = jnp.int32


def _sc_mesh():
    return plsc.VectorSubcoreMesh(core_axis_name="c", subcore_axis_name="s")


def _zero_rows(buf, nrows):
    z16 = jnp.zeros((16,), _f32)

    def body(i, c):
        buf[i, :] = z16
        return c

    lax.fori_loop(0, nrows, body, 0)


def _fill_flat(buf, nvec, value):
    v16 = jnp.full((16,), value, _f32)

    def body(i, c):
        buf[pl.ds(i * 16, 16)] = v16
        return c

    lax.fori_loop(0, nvec, body, 0)


def _agg_body(srcs, dsts, eattr, sums, cnts,
              zrows, czero, idx_v, rows_v, ones_v, acc, csh, lsem):
    cid = lax.axis_index("c")
    sid = lax.axis_index("s")
    _zero_rows(zrows, ZR)
    _fill_flat(czero, CZ // 16, 0.0)
    _fill_flat(ones_v, CH_A // 16, 1.0)
    r0 = sid * RPT
    for j in range(34):
        pltpu.sync_copy(zrows, acc.at[pl.ds(r0 + j * ZR, ZR)])
    for j in range(2):
        pltpu.sync_copy(czero, csh.at[pl.ds(r0 + j * CZ, CZ)])
    plsc.subcore_barrier()

    e_base = sid * (EE // NS)
    NG_A = (EE // NS) // CH_A

    def issue_loads(g, b):
        e0 = e_base + g * CH_A

        @pl.when(cid == 0)
        def _():
            pltpu.async_copy(dsts.at[pl.ds(e0, CH_A)], idx_v.at[b], lsem.at[b])

        @pl.when(cid == 1)
        def _():
            pltpu.async_copy(srcs.at[pl.ds(e0, CH_A)], idx_v.at[b], lsem.at[b])

        pltpu.async_copy(eattr.at[pl.ds(e0, CH_A)], rows_v.at[b], lsem.at[b])

    issue_loads(0, 0)
    issue_loads(1, 1)

    def step(g, c):
        b = lax.rem(g, 2)
        pltpu.make_async_copy(srcs.at[pl.ds(0, CH_A)], idx_v.at[b],
                              lsem.at[b]).wait()
        pltpu.make_async_copy(eattr.at[pl.ds(0, CH_A)], rows_v.at[b],
                              lsem.at[b]).wait()
        pltpu.sync_copy(rows_v.at[b], acc.at[idx_v.at[b]], add=True)
        pltpu.sync_copy(ones_v, csh.at[idx_v.at[b]], add=True)

        @pl.when(g + 2 < NG_A)
        def _():
            issue_loads(g + 2, b)

        return c

    lax.fori_loop(0, NG_A, step, 0)
    plsc.subcore_barrier()
    for j in range(34):
        pltpu.sync_copy(acc.at[pl.ds(r0 + j * ZR, ZR)], zrows)
        pltpu.sync_copy(zrows, sums.at[cid, pl.ds(r0 + j * ZR, ZR)])
    for j in range(2):
        pltpu.sync_copy(csh.at[pl.ds(r0 + j * CZ, CZ)], czero)
        pltpu.sync_copy(czero, cnts.at[pl.ds(cid * NPAD + r0 + j * CZ, CZ)])


_agg = pl.kernel(
    _agg_body,
    out_type=(jax.ShapeDtypeStruct((NC, NPAD, FF), _f32),
              jax.ShapeDtypeStruct((NC * NPAD,), _f32)),
    mesh=_sc_mesh(),
    compiler_params=pltpu.CompilerParams(use_tc_tiling_on_sc=False),
    scratch_types=[
        pltpu.VMEM((ZR, FF), _f32),
        pltpu.VMEM((CZ,), _f32),
        pltpu.VMEM((2, CH_A), _i32),
        pltpu.VMEM((2, CH_A, FF), _f32),
        pltpu.VMEM((CH_A,), _f32),
        pltpu.VMEM_SHARED((NPAD, FF), _f32),
        pltpu.VMEM_SHARED((NPAD,), _f32),
        pltpu.SemaphoreType.DMA((2,)),
    ],
)


def _spmm_body(srcs, dsts, h, parts,
               zrows, idx_s, idx_d, rows_v, acc, sem):
    cid = lax.axis_index("c")
    sid = lax.axis_index("s")
    _zero_rows(zrows, ZR)
    r0 = sid * RPT
    for j in range(34):
        pltpu.sync_copy(zrows, acc.at[pl.ds(r0 + j * ZR, ZR)])
    plsc.subcore_barrier()

    wid = cid * NS + sid
    e_base = wid * (EE // NW)
    NG_C = (EE // NW) // CH_C

    def load_idx(g, b):
        e0 = e_base + g * CH_C
        pltpu.sync_copy(srcs.at[pl.ds(e0, CH_C)], idx_s.at[b])
        pltpu.sync_copy(dsts.at[pl.ds(e0, CH_C)], idx_d.at[b])

    load_idx(0, 0)
    pltpu.async_copy(h.at[idx_s.at[0]], rows_v.at[0], sem.at[0])
    load_idx(1, 1)

    def step(g, c):
        b = lax.rem(g, 2)
        ob = 1 - b

        @pl.when(g + 1 < NG_C)
        def _():
            pltpu.async_copy(h.at[idx_s.at[ob]], rows_v.at[ob], sem.at[ob])

        pltpu.make_async_copy(h.at[idx_s.at[b]], rows_v.at[b],
                              sem.at[b]).wait()
        pltpu.sync_copy(rows_v.at[b], acc.at[idx_d.at[b]], add=True)

        @pl.when(g + 2 < NG_C)
        def _():
            load_idx(g + 2, b)

        return c

    lax.fori_loop(0, NG_C, step, 0)
    plsc.subcore_barrier()
    for j in range(34):
        pltpu.sync_copy(acc.at[pl.ds(r0 + j * ZR, ZR)], zrows)
        pltpu.sync_copy(zrows, parts.at[cid, pl.ds(r0 + j * ZR, ZR)])


_spmm = pl.kernel(
    _spmm_body,
    out_type=jax.ShapeDtypeStruct((NC, NPAD, FF), _f32),
    mesh=_sc_mesh(),
    compiler_params=pltpu.CompilerParams(use_tc_tiling_on_sc=False),
    scratch_types=[
        pltpu.VMEM((ZR, FF), _f32),
        pltpu.VMEM((2, CH_C), _i32),
        pltpu.VMEM((2, CH_C), _i32),
        pltpu.VMEM((2, CH_C, FF), _f32),
        pltpu.VMEM_SHARED((NPAD, FF), _f32),
        pltpu.SemaphoreType.DMA((2,)),
    ],
)


def _edge_body(srcs, dsts, na, u, v, embA, embB, logits,
               s_v, d_v, rowsA, rowsB, ug, vg, lg, ustage, ush, vsh, sem):
    cid = lax.axis_index("c")
    sid = lax.axis_index("s")
    r0 = sid * RPT
    pltpu.sync_copy(u.at[pl.ds(r0, RPT)], ustage)
    pltpu.sync_copy(ustage, ush.at[pl.ds(r0, RPT)])
    pltpu.sync_copy(v.at[pl.ds(r0, RPT)], ustage)
    pltpu.sync_copy(ustage, vsh.at[pl.ds(r0, RPT)])
    plsc.subcore_barrier()

    wid = cid * NS + sid
    e_base = wid * (EE // NW)
    NG_E = (EE // NW) // CH_E

    def load_idx(g, b):
        e0 = e_base + g * CH_E
        pltpu.sync_copy(srcs.at[pl.ds(e0, CH_E)], s_v.at[b])
        pltpu.sync_copy(dsts.at[pl.ds(e0, CH_E)], d_v.at[b])

    def fire(b):
        pltpu.async_copy(na.at[s_v.at[b]], rowsA.at[b], sem.at[b, 0])
        pltpu.async_copy(na.at[d_v.at[b]], rowsB.at[b], sem.at[b, 1])
        pltpu.async_copy(ush.at[s_v.at[b]], ug.at[b], sem.at[b, 2])
        pltpu.async_copy(vsh.at[d_v.at[b]], vg.at[b], sem.at[b, 3])

    load_idx(0, 0)
    fire(0)
    load_idx(1, 1)

    def step(g, c):
        e0 = e_base + g * CH_E
        b = lax.rem(g, 2)
        ob = 1 - b

        @pl.when(g + 1 < NG_E)
        def _():
            fire(ob)

        pltpu.make_async_copy(na.at[s_v.at[b]], rowsA.at[b], sem.at[b, 0]).wait()
        pltpu.make_async_copy(na.at[d_v.at[b]], rowsB.at[b], sem.at[b, 1]).wait()
        pltpu.make_async_copy(ush.at[s_v.at[b]], ug.at[b], sem.at[b, 2]).wait()
        pltpu.make_async_copy(vsh.at[d_v.at[b]], vg.at[b], sem.at[b, 3]).wait()
        pltpu.sync_copy(rowsA.at[b], embA.at[pl.ds(e0, CH_E)])
        pltpu.sync_copy(rowsB.at[b], embB.at[pl.ds(e0, CH_E)])

        def cstep(j, cc):
            lg[pl.ds(j * 16, 16)] = (ug.at[b])[pl.ds(j * 16, 16)] + (vg.at[b])[pl.ds(j * 16, 16)]
            return cc

        lax.fori_loop(0, CH_E // 16, cstep, 0)
        pltpu.sync_copy(lg, logits.at[pl.ds(e0, CH_E)])

        @pl.when(g + 2 < NG_E)
        def _():
            load_idx(g + 2, b)

        return c

    lax.fori_loop(0, NG_E, step, 0)


_edge = pl.kernel(
    _edge_body,
    out_type=(jax.ShapeDtypeStruct((EE, FF), _f32),
              jax.ShapeDtypeStruct((EE, FF), _f32),
              jax.ShapeDtypeStruct((EE,), _f32)),
    mesh=_sc_mesh(),
    compiler_params=pltpu.CompilerParams(use_tc_tiling_on_sc=False),
    scratch_types=[
        pltpu.VMEM((2, CH_E), _i32),
        pltpu.VMEM((2, CH_E), _i32),
        pltpu.VMEM((2, CH_E, FF), _f32),
        pltpu.VMEM((2, CH_E, FF), _f32),
        pltpu.VMEM((2, CH_E), _f32),
        pltpu.VMEM((2, CH_E), _f32),
        pltpu.VMEM((CH_E,), _f32),
        pltpu.VMEM((RPT,), _f32),
        pltpu.VMEM_SHARED((NPAD,), _f32),
        pltpu.VMEM_SHARED((NPAD,), _f32),
        pltpu.SemaphoreType.DMA((2, 4)),
    ],
)


BR = 2000
GRID = NN // BR


def _b0_kernel(sd, ss, cd, cs, wa, wb, b2, h0, ea_out):
    cd_ = jnp.maximum(cd[0], 1.0)
    cs_ = jnp.maximum(cs[0], 1.0)
    na = sd[0] / cd_
    ea = ss[0] / cs_
    h = jnp.dot(na, wa[...], preferred_element_type=_f32)
    h = h + jnp.dot(ea, wb[...], preferred_element_type=_f32)
    h0[...] = jnp.maximum(h + b2[0:1, :], 0.0)
    ea_out[...] = ea


def _db_kernel(p0, p1, cd, h0, wna, wnb, bn2, wea, web, be2, ea, h1_out):
    cd_ = jnp.maximum(cd[0], 1.0)
    nbr = (p0[0] + p1[0]) / cd_
    na1 = jnp.dot(h0[...], wna[...], preferred_element_type=_f32)
    na1 = na1 + jnp.dot(nbr, wnb[...], preferred_element_type=_f32)
    na1 = jnp.maximum(na1 + bn2[0:1, :], 0.0)
    h1 = jnp.dot(na1, wea[...], preferred_element_type=_f32)
    h1 = h1 + jnp.dot(ea[...], web[...], preferred_element_type=_f32)
    h1_out[...] = jnp.maximum(h1 + be2[0:1, :], 0.0)


def _d1_kernel(p0, p1, cd, h1, wna, wnb, bn2, wuv, buv, na2_out, u_out, v_out):
    cd_ = jnp.maximum(cd[0], 1.0)
    nbr = (p0[0] + p1[0]) / cd_
    na2 = jnp.dot(h1[...], wna[...], preferred_element_type=_f32)
    na2 = na2 + jnp.dot(nbr, wnb[...], preferred_element_type=_f32)
    na2 = jnp.maximum(na2 + bn2[0:1, :], 0.0)
    na2_out[...] = na2
    uv = jnp.dot(na2, wuv[...], preferred_element_type=_f32) + buv[0:1, :]
    u_out[...] = uv[:, 0:1]
    v_out[...] = uv[:, 1:2]


def _node_spec():
    return pl.BlockSpec((1, BR, FF), lambda i: (0, i, 0))


def _node_spec1():
    return pl.BlockSpec((1, BR, FF), lambda i: (1, i, 0))


def _cnt_spec(row):
    return pl.BlockSpec((1, BR, 1), lambda i, r=row: (r, i, 0))


def _w_spec(shape):
    return pl.BlockSpec(shape, lambda i: tuple(0 for _ in shape))


def _out_spec(width):
    return pl.BlockSpec((BR, width), lambda i: (i, 0))


_b0_call = pl.pallas_call(
    _b0_kernel,
    grid=(GRID,),
    in_specs=[_node_spec(), _node_spec1(), _cnt_spec(0), _cnt_spec(1),
              _w_spec((FF, FF)), _w_spec((FF, FF)), _w_spec((8, FF))],
    out_specs=[_out_spec(FF), _out_spec(FF)],
    out_shape=(jax.ShapeDtypeStruct((NN, FF), _f32),
               jax.ShapeDtypeStruct((NN, FF), _f32)),
)

_db_call = pl.pallas_call(
    _db_kernel,
    grid=(GRID,),
    in_specs=[_node_spec(), _node_spec1(), _cnt_spec(0), _out_spec(FF),
              _w_spec((FF, FF)), _w_spec((FF, FF)), _w_spec((8, FF)),
              _w_spec((FF, FF)), _w_spec((FF, FF)), _w_spec((8, FF)),
              _out_spec(FF)],
    out_specs=[_out_spec(FF)],
    out_shape=(jax.ShapeDtypeStruct((NN, FF), _f32),),
)

_d1_call = pl.pallas_call(
    _d1_kernel,
    grid=(GRID,),
    in_specs=[_node_spec(), _node_spec1(), _cnt_spec(0), _out_spec(FF),
              _w_spec((FF, FF)), _w_spec((FF, FF)), _w_spec((8, FF)),
              _w_spec((FF, 2)), _w_spec((8, 2))],
    out_specs=[_out_spec(FF), _out_spec(1), _out_spec(1)],
    out_shape=(jax.ShapeDtypeStruct((NN, FF), _f32),
               jax.ShapeDtypeStruct((NN, 1), _f32),
               jax.ShapeDtypeStruct((NN, 1), _f32)),
)


BC = 3200
TGRID = EE // BC


def _emb_tpose_kernel(ea_, eb_, et_out):
    at = jnp.transpose(ea_[...], (1, 0))
    bt = jnp.transpose(eb_[...], (1, 0))
    et_out[...] = jnp.concatenate([at, bt], axis=0)


_emb_tpose = pl.pallas_call(
    _emb_tpose_kernel,
    grid=(TGRID,),
    in_specs=[pl.BlockSpec((BC, FF), lambda i: (i, 0)),
              pl.BlockSpec((BC, FF), lambda i: (i, 0))],
    out_specs=[pl.BlockSpec((2 * FF, BC), lambda i: (0, i))],
    out_shape=(jax.ShapeDtypeStruct((2 * FF, EE), _f32),),
)


def kernel(edge_attr, edge_index, We0, be0, Wn0, bn0, We1, be1, Wn1, bn1, Wf, bf):
    srcs = edge_index[0]
    dsts = edge_index[1]
    sums, cnts = _agg(srcs, dsts, edge_attr)
    cnts3 = cnts.reshape(NC, NPAD, 1)

    def split_t(W):
        return W[:, :FF].T, W[:, FF:].T

    def b2(b):
        return jnp.broadcast_to(b.reshape(1, -1), (8, b.shape[0]))

    WeA0, WeB0 = split_t(We0)
    WnA0, WnB0 = split_t(Wn0)
    WeA1, WeB1 = split_t(We1)
    WnA1, WnB1 = split_t(Wn1)
    Wuv = jnp.stack([Wf[0, :FF], Wf[0, FF:]], axis=1)
    buv = jnp.broadcast_to(jnp.stack([bf[0], jnp.zeros((), _f32)]).reshape(1, 2), (8, 2))

    h0, eagg = _b0_call(sums, sums, cnts3, cnts3, WeA0, WeB0, b2(be0))
    parts0 = _spmm(srcs, dsts, h0)
    (h1,) = _db_call(parts0, parts0, cnts3, h0, WnA0, WnB0, b2(bn0),
                     WeA1, WeB1, b2(be1), eagg)
    parts1 = _spmm(srcs, dsts, h1)
    na2, u, v = _d1_call(parts1, parts1, cnts3, h1, WnA1, WnB1, b2(bn1),
                         Wuv, buv)

    pad = jnp.zeros((NPAD - NN,), _f32)
    u_p = jnp.concatenate([u.reshape(-1), pad])
    v_p = jnp.concatenate([v.reshape(-1), pad])

    embA, embB, logits = _edge(srcs, dsts, na2, u_p, v_p)
    (emb_t,) = _emb_tpose(embA, embB)
    edge_embeddings = jnp.transpose(emb_t, (1, 0))
    return logits, edge_embeddings, na2
